# baseline (speedup 1.0000x reference)
import functools

import jax
import jax.numpy as jnp
from jax.experimental import pallas as pl
from jax.experimental.pallas import tpu as pltpu

_B, _N0 = 16, 4096
_EPS = 1e-5


def _rb(x):
    return jax.lax.bitcast_convert_type(
        jax.lax.bitcast_convert_type(x, jnp.int32), jnp.float32)


def _fps_body(xyz_ref, nxyz_ref, dists_ref, *, N, S):
    X = xyz_ref[0]
    Y = xyz_ref[1]
    Z = xyz_ref[2]
    iota = jax.lax.broadcasted_iota(jnp.int32, (_B, N), 1)
    dists_ref[...] = jnp.full((_B, N), 1e10, jnp.float32)

    def fps_step(s, far):
        sel = iota == far
        cx = jnp.sum(jnp.where(sel, X, 0.0), axis=1, keepdims=True)
        cy = jnp.sum(jnp.where(sel, Y, 0.0), axis=1, keepdims=True)
        cz = jnp.sum(jnp.where(sel, Z, 0.0), axis=1, keepdims=True)
        dx = X - cx
        dy = Y - cy
        dz = Z - cz
        d = _rb(dx * dx) + _rb(dy * dy) + _rb(dz * dz)
        dm = jnp.minimum(dists_ref[...], d)
        dists_ref[...] = dm
        cvec = jnp.concatenate([cx, cy, cz], axis=1)
        nxyz_ref[:, pl.ds(s, 1), :] = cvec.reshape(_B, 1, 3)
        m = jnp.max(dm, axis=1, keepdims=True)
        nxt = jnp.min(jnp.where(dm == m, iota, N), axis=1, keepdims=True)
        return nxt

    jax.lax.fori_loop(0, S, fps_step, jnp.zeros((_B, 1), jnp.int32),
                      unroll=False)


def _fps(xyz_cbn, *, N, S):
    body = functools.partial(_fps_body, N=N, S=S)
    return pl.pallas_call(
        body,
        grid=(1,),
        in_specs=[pl.BlockSpec((3, _B, N), lambda i: (0, 0, 0))],
        out_specs=pl.BlockSpec((_B, S, 3), lambda i: (0, 0, 0)),
        out_shape=jax.ShapeDtypeStruct((_B, S, 3), jnp.float32),
        scratch_shapes=[pltpu.VMEM((_B, N), jnp.float32)],
    )(xyz_cbn)


def _group_body(xyz_ref, nxyz_in_ref, tcat_ref, g_ref,
                *, N, S, ns, C_pad, r2, KC):
    X = xyz_ref[0, 0:1, :]
    Y = xyz_ref[0, 1:2, :]
    Z = xyz_ref[0, 2:3, :]
    nxyz = nxyz_in_ref[0]
    sx = nxyz[:, 0:1]
    sy = nxyz[:, 1:2]
    sz = nxyz[:, 2:3]
    dxs = sx - X
    dys = sy - Y
    dzs = sz - Z
    sq = _rb(dxs * dxs) + _rb(dys * dys) + _rb(dzs * dzs)
    mask = sq <= r2
    m = jnp.where(mask, 1.0, 0.0)
    lane = jax.lax.broadcasted_iota(jnp.int32, (S, N), 1)
    lane_mod = jax.lax.bitwise_and(lane, 127)
    w = m
    k = 1
    while k < min(128, N):
        r = pltpu.roll(w, k, 1)
        w = w + jnp.where(lane_mod >= k, r, 0.0)
        k *= 2
    if N > 128:
        nblk = N // 128
        ii = jax.lax.broadcasted_iota(jnp.int32, (nblk, N), 0)
        jj = jax.lax.broadcasted_iota(jnp.int32, (nblk, N), 1)
        cum_sel = jnp.where(ii < jax.lax.shift_right_logical(jj, 7),
                            1.0, 0.0)
        bd = jnp.where(jax.lax.shift_right_logical(
            jax.lax.broadcasted_iota(jnp.int32, (N, nblk), 0), 7)
            == jax.lax.broadcasted_iota(jnp.int32, (N, nblk), 1),
            1.0, 0.0)
        bs = jnp.dot(m, bd, preferred_element_type=jnp.float32)
        c = w + jnp.dot(bs, cum_sel, preferred_element_type=jnp.float32)
    else:
        c = w
    cm = jnp.where(mask, c, 0.0)
    count = c[:, N - 1:N]
    nxyz_pad = jnp.concatenate(
        [nxyz, jnp.zeros((S, C_pad - 3), jnp.float32)], axis=1)
    tcat = tcat_ref[0]

    cm_t = jnp.concatenate([cm] * KC, axis=0)
    slot = (jax.lax.broadcasted_iota(jnp.int32, (KC * S, 1), 0)
            // S).astype(jnp.float32)

    def chunk_rows(base):
        kv = slot + (base + 1).astype(jnp.float32)
        oh = jnp.where(cm_t == kv, 1.0, 0.0)
        rows3 = jnp.dot(oh, tcat, preferred_element_type=jnp.float32)
        return (rows3[:, 0:C_pad] + rows3[:, C_pad:2 * C_pad]
                ) + rows3[:, 2 * C_pad:3 * C_pad]

    rows = chunk_rows(jnp.int32(0))
    g0 = rows[0:S] - nxyz_pad
    g_ref[0, 0, :, :] = g0
    for i in range(1, KC):
        out = rows[i * S:(i + 1) * S] - nxyz_pad
        out = jnp.where(count > float(i), out, g0)
        g_ref[0, i, :, :] = out

    if ns > KC:
        def sel_chunk(kb, _):
            base = kb * KC
            rows = chunk_rows(base)
            g0v = g_ref[0, 0, :, :]
            for i in range(KC):
                out = rows[i * S:(i + 1) * S] - nxyz_pad
                out = jnp.where(count > (base + i).astype(jnp.float32),
                                out, g0v)
                g_ref[0, base + i, :, :] = out
            return 0

        jax.lax.fori_loop(1, ns // KC, sel_chunk, 0, unroll=False)


def _group(xyz_t, nxyz, tcat, *, N, S, ns, C_pad, r2):
    KC = ns if S * N * ns * 4 <= (4 << 20) else 16
    body = functools.partial(_group_body, N=N, S=S, ns=ns, C_pad=C_pad,
                             r2=r2, KC=KC)
    return pl.pallas_call(
        body,
        grid=(_B,),
        in_specs=[
            pl.BlockSpec((1, 3, N), lambda b: (b, 0, 0)),
            pl.BlockSpec((1, S, 3), lambda b: (b, 0, 0)),
            pl.BlockSpec((1, N, 3 * C_pad), lambda b: (b, 0, 0)),
        ],
        out_specs=pl.BlockSpec((1, ns, S, C_pad), lambda b: (b, 0, 0, 0)),
        out_shape=jax.ShapeDtypeStruct((_B, ns, S, C_pad), jnp.float32),
    )(xyz_t, nxyz, tcat)


def _mlp_body(x_ref, w_ref, bias_ref, mu_ref, var_ref, gamma_ref, beta_ref,
              z_ref, ssum_ref, *, normalize, cnt, stats_inline=False,
              fold=1):
    x = x_ref[0]
    if normalize:
        if stats_inline:
            def _chan_stats(v):
                s = jnp.sum(v, axis=0, keepdims=True)
                if fold > 1:
                    C = s.shape[1] // fold
                    sc = s[:, 0:C]
                    for i in range(1, fold):
                        sc = sc + s[:, i * C:(i + 1) * C]
                    s = jnp.concatenate([sc] * fold, axis=1)
                return s / cnt
            mu = _chan_stats(x)
            dx = x - mu
            var = _chan_stats(dx * dx)
        else:
            mu = mu_ref[...] / cnt
            var = var_ref[...] / cnt
        xn = (x - mu) * jax.lax.rsqrt(var + _EPS)
        x = jnp.maximum(xn * gamma_ref[...] + beta_ref[...], 0.0)
    z = jnp.dot(x, w_ref[...], preferred_element_type=jnp.float32)
    z = z + bias_ref[...]
    z_ref[0] = z

    @pl.when(pl.program_id(0) == 0)
    def _():
        ssum_ref[...] = jnp.zeros_like(ssum_ref)

    ssum_ref[...] += jnp.sum(z, axis=0, keepdims=True)


def _mlp(x, w, bias, mu, var, gamma, beta, *, normalize, cnt,
         stats_inline=False, fold=1):
    G, R, C_in = x.shape
    C_out = w.shape[1]
    body = functools.partial(_mlp_body, normalize=normalize, cnt=cnt,
                             stats_inline=stats_inline, fold=fold)
    return pl.pallas_call(
        body,
        grid=(G,),
        in_specs=[
            pl.BlockSpec((1, R, C_in), lambda b: (b, 0, 0)),
            pl.BlockSpec((C_in, C_out), lambda b: (0, 0)),
            pl.BlockSpec((1, C_out), lambda b: (0, 0)),
            pl.BlockSpec((1, C_in), lambda b: (0, 0)),
            pl.BlockSpec((1, C_in), lambda b: (0, 0)),
            pl.BlockSpec((1, C_in), lambda b: (0, 0)),
            pl.BlockSpec((1, C_in), lambda b: (0, 0)),
        ],
        out_specs=[
            pl.BlockSpec((1, R, C_out), lambda b: (b, 0, 0)),
            pl.BlockSpec((1, C_out), lambda b: (0, 0)),
        ],
        out_shape=[
            jax.ShapeDtypeStruct((G, R, C_out), jnp.float32),
            jax.ShapeDtypeStruct((1, C_out), jnp.float32),
        ],
    )(x, w, bias, mu, var, gamma, beta)


def _var_body(z_ref, ssum_ref, svar_ref, *, cnt):
    z = z_ref[0]
    mu = ssum_ref[...] / cnt
    d = z - mu
    sv = jnp.sum(d * d, axis=0, keepdims=True)

    @pl.when(pl.program_id(0) == 0)
    def _():
        svar_ref[...] = jnp.zeros_like(svar_ref)

    svar_ref[...] += sv


def _var(z, ssum, *, cnt):
    G, R, C = z.shape
    body = functools.partial(_var_body, cnt=cnt)
    return pl.pallas_call(
        body,
        grid=(G,),
        in_specs=[
            pl.BlockSpec((1, R, C), lambda b: (b, 0, 0)),
            pl.BlockSpec((1, C), lambda b: (0, 0)),
        ],
        out_specs=pl.BlockSpec((1, C), lambda b: (0, 0)),
        out_shape=jax.ShapeDtypeStruct((1, C), jnp.float32),
    )(z, ssum)


def _norm_max_body(z_ref, mu_ref, var_ref, gamma_ref, beta_ref, out_ref,
                   *, cnt):
    z = z_ref[0]
    mu = mu_ref[...] / cnt
    var = var_ref[...] / cnt
    y = (z - mu) * jax.lax.rsqrt(var + _EPS)
    y = jnp.maximum(y * gamma_ref[...] + beta_ref[...], 0.0)
    out_ref[0] = jnp.max(y, axis=0)


def _norm_max_whole_body(z_ref, gamma_ref, beta_ref, out_ref, *, ns, cnt,
                         fold=1):
    B, NS, S, C = z_ref.shape
    zf = z_ref[...].reshape(B * NS * S, C)

    def _chan_stats(v):
        s = jnp.sum(v, axis=0, keepdims=True)
        if fold > 1:
            Cc = s.shape[1] // fold
            sc = s[:, 0:Cc]
            for i in range(1, fold):
                sc = sc + s[:, i * Cc:(i + 1) * Cc]
            s = jnp.concatenate([sc] * fold, axis=1)
        return s / cnt

    mu = _chan_stats(zf)
    d = zf - mu
    var = _chan_stats(d * d)
    y = d * jax.lax.rsqrt(var + _EPS)
    y = jnp.maximum(y * gamma_ref[...] + beta_ref[...], 0.0)
    y4 = y.reshape(B, NS, S, C)
    acc = y4[:, 0]
    for k in range(1, ns):
        acc = jnp.maximum(acc, y4[:, k])
    out_ref[...] = acc


def _norm_max_whole(z, gamma, beta, *, ns, cnt, fold=1):
    B, NS, S, C = z.shape
    body = functools.partial(_norm_max_whole_body, ns=ns, cnt=cnt, fold=fold)
    return pl.pallas_call(
        body,
        grid=(1,),
        in_specs=[
            pl.BlockSpec((B, NS, S, C), lambda i: (0, 0, 0, 0)),
            pl.BlockSpec((1, C), lambda i: (0, 0)),
            pl.BlockSpec((1, C), lambda i: (0, 0)),
        ],
        out_specs=pl.BlockSpec((B, S, C), lambda i: (0, 0, 0)),
        out_shape=jax.ShapeDtypeStruct((B, S, C), jnp.float32),
    )(z, gamma, beta)


def _norm_max(z, ssum, svar, gamma, beta, *, ns, S, cnt):
    C = z.shape[-1]
    body = functools.partial(_norm_max_body, cnt=cnt)
    return pl.pallas_call(
        body,
        grid=(_B,),
        in_specs=[
            pl.BlockSpec((1, ns, S, C), lambda b: (b, 0, 0, 0)),
            pl.BlockSpec((1, C), lambda b: (0, 0)),
            pl.BlockSpec((1, C), lambda b: (0, 0)),
            pl.BlockSpec((1, C), lambda b: (0, 0)),
            pl.BlockSpec((1, C), lambda b: (0, 0)),
        ],
        out_specs=pl.BlockSpec((1, S, C), lambda b: (b, 0, 0)),
        out_shape=jax.ShapeDtypeStruct((_B, S, C), jnp.float32),
    )(z, ssum, svar, gamma, beta)


def _pad_lanes(x, C_pad):
    c = x.shape[-1]
    if c == C_pad:
        return x
    pad = [(0, 0)] * (x.ndim - 1) + [(0, C_pad - c)]
    return jnp.pad(x, pad)


def _sa_module(xyz, feat_rows, layers, *, N, S, ns, C_pad, radius):
    if feat_rows is None:
        table = jnp.concatenate([xyz, xyz, xyz], axis=-1)
    else:
        table = jnp.concatenate([xyz, feat_rows], axis=-1)
    C_in = table.shape[-1]
    table = _pad_lanes(table, C_pad)
    def _trunc16(x):
        xi = jax.lax.bitcast_convert_type(x, jnp.int32)
        return jax.lax.bitcast_convert_type(
            jax.lax.bitwise_and(xi, jnp.int32(-65536)), jnp.float32)

    t_hi = _trunc16(table)
    r = table - t_hi
    t_mid = _trunc16(r)
    tcat = jnp.concatenate([t_hi, t_mid, r - t_mid], axis=-1)
    xyz_t = jnp.transpose(xyz, (0, 2, 1))
    nxyz = _fps(jnp.transpose(xyz, (2, 0, 1)), N=N, S=S)
    g = _group(xyz_t, nxyz, tcat, N=N, S=S, ns=ns,
               C_pad=C_pad, r2=radius ** 2)
    R = ns * S
    cnt = float(_B * R)
    F = max(1, 128 // C_pad)
    (w1, b1, g1, be1), (w2, b2, g2, be2) = layers
    w1t = jnp.pad(jnp.transpose(w1), ((0, C_pad - C_in), (0, 0)))
    w2t = jnp.transpose(w2)
    c2 = w2.shape[0]
    if F > 1:
        eye = jnp.eye(F, dtype=jnp.float32)
        w1t = jnp.kron(eye, w1t)
        w2t = jnp.kron(eye, w2t)

    def _tile(v):
        return jnp.tile(v.reshape(1, -1), (1, F))

    x = g.reshape(_B, ns, S // F, F * C_pad).reshape(
        1, _B * R // F, F * C_pad)
    dummy1 = jnp.zeros((1, F * C_pad), jnp.float32)
    z1, _ = _mlp(x, w1t, _tile(b1), dummy1, dummy1, dummy1,
                 dummy1, normalize=False, cnt=cnt)
    z2, _ = _mlp(z1, w2t, _tile(b2), dummy1,
                 dummy1, _tile(g1), _tile(be1),
                 normalize=True, cnt=cnt, stats_inline=True, fold=F)
    out = _norm_max_whole(z2.reshape(_B, ns, S // F, F * c2), _tile(g2),
                          _tile(be2), ns=ns, cnt=cnt, fold=F)
    out = out.reshape(_B, S, c2)
    return nxyz, out


def kernel(pc, params):
    xyz = pc[:, :, :3]
    nxyz1, f1 = _sa_module(xyz, None, params[0],
                           N=_N0, S=128, ns=32, C_pad=16, radius=0.02)
    nxyz2, f2 = _sa_module(nxyz1, f1, params[1],
                           N=128, S=64, ns=32, C_pad=128, radius=0.04)
    _, f3 = _sa_module(nxyz2, f2, params[2],
                       N=64, S=32, ns=16, C_pad=256, radius=0.08)
    return jnp.transpose(f3, (0, 2, 1))


# speedup vs baseline: 1.0065x; 1.0065x over previous
import functools

import jax
import jax.numpy as jnp
from jax.experimental import pallas as pl
from jax.experimental.pallas import tpu as pltpu

_B, _N0 = 16, 4096
_EPS = 1e-5


def _rb(x):
    return jax.lax.bitcast_convert_type(
        jax.lax.bitcast_convert_type(x, jnp.int32), jnp.float32)


def _fps_body(xyz_ref, nxyz_ref, dists_ref, *, N, S):
    X = xyz_ref[0]
    Y = xyz_ref[1]
    Z = xyz_ref[2]
    iota = jax.lax.broadcasted_iota(jnp.int32, (_B, N), 1)
    dists_ref[...] = jnp.full((_B, N), 1e10, jnp.float32)

    def fps_step(s, far):
        sel = iota == far
        cx = jnp.sum(jnp.where(sel, X, 0.0), axis=1, keepdims=True)
        cy = jnp.sum(jnp.where(sel, Y, 0.0), axis=1, keepdims=True)
        cz = jnp.sum(jnp.where(sel, Z, 0.0), axis=1, keepdims=True)
        dx = X - cx
        dy = Y - cy
        dz = Z - cz
        d = _rb(dx * dx) + _rb(dy * dy) + _rb(dz * dz)
        dm = jnp.minimum(dists_ref[...], d)
        dists_ref[...] = dm
        cvec = jnp.concatenate([cx, cy, cz], axis=1)
        nxyz_ref[:, pl.ds(s, 1), :] = cvec.reshape(_B, 1, 3)
        m = jnp.max(dm, axis=1, keepdims=True)
        nxt = jnp.min(jnp.where(dm == m, iota, N), axis=1, keepdims=True)
        return nxt

    jax.lax.fori_loop(0, S, fps_step, jnp.zeros((_B, 1), jnp.int32),
                      unroll=False)


def _fps(xyz_cbn, *, N, S):
    body = functools.partial(_fps_body, N=N, S=S)
    return pl.pallas_call(
        body,
        grid=(1,),
        in_specs=[pl.BlockSpec((3, _B, N), lambda i: (0, 0, 0))],
        out_specs=pl.BlockSpec((_B, S, 3), lambda i: (0, 0, 0)),
        out_shape=jax.ShapeDtypeStruct((_B, S, 3), jnp.float32),
        scratch_shapes=[pltpu.VMEM((_B, N), jnp.float32)],
    )(xyz_cbn)


def _group_body(xyz_ref, nxyz_in_ref, tcat_ref, g_ref,
                *, N, S, ns, C_pad, r2, KC):
    X = xyz_ref[0, 0:1, :]
    Y = xyz_ref[0, 1:2, :]
    Z = xyz_ref[0, 2:3, :]
    nxyz = nxyz_in_ref[0]
    sx = nxyz[:, 0:1]
    sy = nxyz[:, 1:2]
    sz = nxyz[:, 2:3]
    dxs = sx - X
    dys = sy - Y
    dzs = sz - Z
    sq = _rb(dxs * dxs) + _rb(dys * dys) + _rb(dzs * dzs)
    mask = sq <= r2
    m = jnp.where(mask, 1.0, 0.0)
    lane = jax.lax.broadcasted_iota(jnp.int32, (S, N), 1)
    lane_mod = jax.lax.bitwise_and(lane, 127)
    w = m
    k = 1
    while k < min(128, N):
        r = pltpu.roll(w, k, 1)
        w = w + jnp.where(lane_mod >= k, r, 0.0)
        k *= 2
    if N > 128:
        nblk = N // 128
        ii = jax.lax.broadcasted_iota(jnp.int32, (nblk, N), 0)
        jj = jax.lax.broadcasted_iota(jnp.int32, (nblk, N), 1)
        cum_sel = jnp.where(ii < jax.lax.shift_right_logical(jj, 7),
                            1.0, 0.0)
        bd = jnp.where(jax.lax.shift_right_logical(
            jax.lax.broadcasted_iota(jnp.int32, (N, nblk), 0), 7)
            == jax.lax.broadcasted_iota(jnp.int32, (N, nblk), 1),
            1.0, 0.0)
        bs = jnp.dot(m, bd, preferred_element_type=jnp.float32)
        c = w + jnp.dot(bs, cum_sel, preferred_element_type=jnp.float32)
    else:
        c = w
    cm = jnp.where(mask, c, 0.0)
    count = c[:, N - 1:N]
    nxyz_pad = jnp.concatenate(
        [nxyz, jnp.zeros((S, C_pad - 3), jnp.float32)], axis=1)
    tcat = tcat_ref[0]

    def chunk_rows(base):
        ohs = [jnp.where(cm == (base + (i + 1)).astype(jnp.float32),
                         1.0, 0.0) for i in range(KC)]
        oh = jnp.concatenate(ohs, axis=0)
        rows3 = jnp.dot(oh, tcat, preferred_element_type=jnp.float32)
        return (rows3[:, 0:C_pad] + rows3[:, C_pad:2 * C_pad]
                ) + rows3[:, 2 * C_pad:3 * C_pad]

    rows = chunk_rows(jnp.int32(0))
    g0 = rows[0:S] - nxyz_pad
    g_ref[0, 0, :, :] = g0
    for i in range(1, KC):
        out = rows[i * S:(i + 1) * S] - nxyz_pad
        out = jnp.where(count > float(i), out, g0)
        g_ref[0, i, :, :] = out

    for kb in range(1, ns // KC):
        base = kb * KC
        rows = chunk_rows(jnp.int32(base))
        for i in range(KC):
            out = rows[i * S:(i + 1) * S] - nxyz_pad
            out = jnp.where(count > float(base + i), out, g0)
            g_ref[0, base + i, :, :] = out


def _group(xyz_t, nxyz, tcat, *, N, S, ns, C_pad, r2):
    KC = ns if S * N * ns * 4 <= (4 << 20) else 16
    body = functools.partial(_group_body, N=N, S=S, ns=ns, C_pad=C_pad,
                             r2=r2, KC=KC)
    return pl.pallas_call(
        body,
        grid=(_B,),
        in_specs=[
            pl.BlockSpec((1, 3, N), lambda b: (b, 0, 0)),
            pl.BlockSpec((1, S, 3), lambda b: (b, 0, 0)),
            pl.BlockSpec((1, N, 3 * C_pad), lambda b: (b, 0, 0)),
        ],
        out_specs=pl.BlockSpec((1, ns, S, C_pad), lambda b: (b, 0, 0, 0)),
        out_shape=jax.ShapeDtypeStruct((_B, ns, S, C_pad), jnp.float32),
    )(xyz_t, nxyz, tcat)


def _mlp_body(x_ref, w_ref, bias_ref, mu_ref, var_ref, gamma_ref, beta_ref,
              z_ref, ssum_ref, *, normalize, cnt, stats_inline=False,
              fold=1):
    x = x_ref[0]
    if normalize:
        if stats_inline:
            def _chan_stats(v):
                s = jnp.sum(v, axis=0, keepdims=True)
                if fold > 1:
                    C = s.shape[1] // fold
                    sc = s[:, 0:C]
                    for i in range(1, fold):
                        sc = sc + s[:, i * C:(i + 1) * C]
                    s = jnp.concatenate([sc] * fold, axis=1)
                return s / cnt
            mu = _chan_stats(x)
            dx = x - mu
            var = _chan_stats(dx * dx)
        else:
            mu = mu_ref[...] / cnt
            var = var_ref[...] / cnt
        xn = (x - mu) * jax.lax.rsqrt(var + _EPS)
        x = jnp.maximum(xn * gamma_ref[...] + beta_ref[...], 0.0)
    z = jnp.dot(x, w_ref[...], preferred_element_type=jnp.float32)
    z = z + bias_ref[...]
    z_ref[0] = z

    @pl.when(pl.program_id(0) == 0)
    def _():
        ssum_ref[...] = jnp.zeros_like(ssum_ref)

    ssum_ref[...] += jnp.sum(z, axis=0, keepdims=True)


def _mlp(x, w, bias, mu, var, gamma, beta, *, normalize, cnt,
         stats_inline=False, fold=1):
    G, R, C_in = x.shape
    C_out = w.shape[1]
    body = functools.partial(_mlp_body, normalize=normalize, cnt=cnt,
                             stats_inline=stats_inline, fold=fold)
    return pl.pallas_call(
        body,
        grid=(G,),
        in_specs=[
            pl.BlockSpec((1, R, C_in), lambda b: (b, 0, 0)),
            pl.BlockSpec((C_in, C_out), lambda b: (0, 0)),
            pl.BlockSpec((1, C_out), lambda b: (0, 0)),
            pl.BlockSpec((1, C_in), lambda b: (0, 0)),
            pl.BlockSpec((1, C_in), lambda b: (0, 0)),
            pl.BlockSpec((1, C_in), lambda b: (0, 0)),
            pl.BlockSpec((1, C_in), lambda b: (0, 0)),
        ],
        out_specs=[
            pl.BlockSpec((1, R, C_out), lambda b: (b, 0, 0)),
            pl.BlockSpec((1, C_out), lambda b: (0, 0)),
        ],
        out_shape=[
            jax.ShapeDtypeStruct((G, R, C_out), jnp.float32),
            jax.ShapeDtypeStruct((1, C_out), jnp.float32),
        ],
    )(x, w, bias, mu, var, gamma, beta)


def _var_body(z_ref, ssum_ref, svar_ref, *, cnt):
    z = z_ref[0]
    mu = ssum_ref[...] / cnt
    d = z - mu
    sv = jnp.sum(d * d, axis=0, keepdims=True)

    @pl.when(pl.program_id(0) == 0)
    def _():
        svar_ref[...] = jnp.zeros_like(svar_ref)

    svar_ref[...] += sv


def _var(z, ssum, *, cnt):
    G, R, C = z.shape
    body = functools.partial(_var_body, cnt=cnt)
    return pl.pallas_call(
        body,
        grid=(G,),
        in_specs=[
            pl.BlockSpec((1, R, C), lambda b: (b, 0, 0)),
            pl.BlockSpec((1, C), lambda b: (0, 0)),
        ],
        out_specs=pl.BlockSpec((1, C), lambda b: (0, 0)),
        out_shape=jax.ShapeDtypeStruct((1, C), jnp.float32),
    )(z, ssum)


def _norm_max_body(z_ref, mu_ref, var_ref, gamma_ref, beta_ref, out_ref,
                   *, cnt):
    z = z_ref[0]
    mu = mu_ref[...] / cnt
    var = var_ref[...] / cnt
    y = (z - mu) * jax.lax.rsqrt(var + _EPS)
    y = jnp.maximum(y * gamma_ref[...] + beta_ref[...], 0.0)
    out_ref[0] = jnp.max(y, axis=0)


def _norm_max_whole_body(z_ref, gamma_ref, beta_ref, out_ref, *, ns, cnt,
                         fold=1):
    B, NS, S, C = z_ref.shape
    zf = z_ref[...].reshape(B * NS * S, C)

    def _chan_stats(v):
        s = jnp.sum(v, axis=0, keepdims=True)
        if fold > 1:
            Cc = s.shape[1] // fold
            sc = s[:, 0:Cc]
            for i in range(1, fold):
                sc = sc + s[:, i * Cc:(i + 1) * Cc]
            s = jnp.concatenate([sc] * fold, axis=1)
        return s / cnt

    mu = _chan_stats(zf)
    d = zf - mu
    var = _chan_stats(d * d)
    y = d * jax.lax.rsqrt(var + _EPS)
    y = jnp.maximum(y * gamma_ref[...] + beta_ref[...], 0.0)
    y4 = y.reshape(B, NS, S, C)
    acc = y4[:, 0]
    for k in range(1, ns):
        acc = jnp.maximum(acc, y4[:, k])
    out_ref[...] = acc


def _norm_max_whole(z, gamma, beta, *, ns, cnt, fold=1):
    B, NS, S, C = z.shape
    body = functools.partial(_norm_max_whole_body, ns=ns, cnt=cnt, fold=fold)
    return pl.pallas_call(
        body,
        grid=(1,),
        in_specs=[
            pl.BlockSpec((B, NS, S, C), lambda i: (0, 0, 0, 0)),
            pl.BlockSpec((1, C), lambda i: (0, 0)),
            pl.BlockSpec((1, C), lambda i: (0, 0)),
        ],
        out_specs=pl.BlockSpec((B, S, C), lambda i: (0, 0, 0)),
        out_shape=jax.ShapeDtypeStruct((B, S, C), jnp.float32),
    )(z, gamma, beta)


def _norm_max(z, ssum, svar, gamma, beta, *, ns, S, cnt):
    C = z.shape[-1]
    body = functools.partial(_norm_max_body, cnt=cnt)
    return pl.pallas_call(
        body,
        grid=(_B,),
        in_specs=[
            pl.BlockSpec((1, ns, S, C), lambda b: (b, 0, 0, 0)),
            pl.BlockSpec((1, C), lambda b: (0, 0)),
            pl.BlockSpec((1, C), lambda b: (0, 0)),
            pl.BlockSpec((1, C), lambda b: (0, 0)),
            pl.BlockSpec((1, C), lambda b: (0, 0)),
        ],
        out_specs=pl.BlockSpec((1, S, C), lambda b: (b, 0, 0)),
        out_shape=jax.ShapeDtypeStruct((_B, S, C), jnp.float32),
    )(z, ssum, svar, gamma, beta)


def _pad_lanes(x, C_pad):
    c = x.shape[-1]
    if c == C_pad:
        return x
    pad = [(0, 0)] * (x.ndim - 1) + [(0, C_pad - c)]
    return jnp.pad(x, pad)


def _sa_module(xyz, feat_rows, layers, *, N, S, ns, C_pad, radius):
    if feat_rows is None:
        table = jnp.concatenate([xyz, xyz, xyz], axis=-1)
    else:
        table = jnp.concatenate([xyz, feat_rows], axis=-1)
    C_in = table.shape[-1]
    table = _pad_lanes(table, C_pad)
    def _trunc16(x):
        xi = jax.lax.bitcast_convert_type(x, jnp.int32)
        return jax.lax.bitcast_convert_type(
            jax.lax.bitwise_and(xi, jnp.int32(-65536)), jnp.float32)

    t_hi = _trunc16(table)
    r = table - t_hi
    t_mid = _trunc16(r)
    tcat = jnp.concatenate([t_hi, t_mid, r - t_mid], axis=-1)
    xyz_t = jnp.transpose(xyz, (0, 2, 1))
    nxyz = _fps(jnp.transpose(xyz, (2, 0, 1)), N=N, S=S)
    g = _group(xyz_t, nxyz, tcat, N=N, S=S, ns=ns,
               C_pad=C_pad, r2=radius ** 2)
    R = ns * S
    cnt = float(_B * R)
    F = max(1, 128 // C_pad)
    (w1, b1, g1, be1), (w2, b2, g2, be2) = layers
    w1t = jnp.pad(jnp.transpose(w1), ((0, C_pad - C_in), (0, 0)))
    w2t = jnp.transpose(w2)
    c2 = w2.shape[0]
    if F > 1:
        eye = jnp.eye(F, dtype=jnp.float32)
        w1t = jnp.kron(eye, w1t)
        w2t = jnp.kron(eye, w2t)

    def _tile(v):
        return jnp.tile(v.reshape(1, -1), (1, F))

    x = g.reshape(_B, ns, S // F, F * C_pad).reshape(
        1, _B * R // F, F * C_pad)
    dummy1 = jnp.zeros((1, F * C_pad), jnp.float32)
    z1, _ = _mlp(x, w1t, _tile(b1), dummy1, dummy1, dummy1,
                 dummy1, normalize=False, cnt=cnt)
    z2, _ = _mlp(z1, w2t, _tile(b2), dummy1,
                 dummy1, _tile(g1), _tile(be1),
                 normalize=True, cnt=cnt, stats_inline=True, fold=F)
    out = _norm_max_whole(z2.reshape(_B, ns, S // F, F * c2), _tile(g2),
                          _tile(be2), ns=ns, cnt=cnt, fold=F)
    out = out.reshape(_B, S, c2)
    return nxyz, out


def kernel(pc, params):
    xyz = pc[:, :, :3]
    nxyz1, f1 = _sa_module(xyz, None, params[0],
                           N=_N0, S=128, ns=32, C_pad=16, radius=0.02)
    nxyz2, f2 = _sa_module(nxyz1, f1, params[1],
                           N=128, S=64, ns=32, C_pad=128, radius=0.04)
    _, f3 = _sa_module(nxyz2, f2, params[2],
                       N=64, S=32, ns=16, C_pad=256, radius=0.08)
    return jnp.transpose(f3, (0, 2, 1))
